# baseline (speedup 1.0000x reference)
import functools

import jax
import jax.numpy as jnp
from jax import lax
from jax.experimental import pallas as pl
from jax.experimental.pallas import tpu as pltpu
from jax.experimental.pallas import tpu_sc as plsc

_VOCAB = 100000
_D = 768
_B = 4
_S = 2048
_HW = 224
_P = 16
_G = _HW // _P
_V = _G * _G
_SEQ = _S + _V
_BLK = 256
_NJ = _SEQ // _BLK + 1

_NW = 32
_TOK = _B * _S
_TPW = _TOK // _NW
_CH = 64
_NCH = _TPW // _CH


_NSEG = _B * 3 * _HW * (_HW // _P)
_SPW = _NSEG // _NW
_SCH = 128
_NSCH = -(-_SPW // _SCH)


def _sc_gather(table, ids, ntok):
    mesh = plsc.VectorSubcoreMesh(core_axis_name="c", subcore_axis_name="s")
    tpw = ntok // _NW
    nch = tpw // _CH

    @functools.partial(
        pl.kernel,
        mesh=mesh,
        out_type=jax.ShapeDtypeStruct((ntok, _D), jnp.float32),
        scratch_types=[
            pltpu.VMEM((tpw,), jnp.int32),
            pltpu.VMEM((_CH, _D), jnp.float32),
            pltpu.VMEM((_CH, _D), jnp.float32),
            pltpu.SemaphoreType.DMA,
            pltpu.SemaphoreType.DMA,
        ],
    )
    def k(table_hbm, ids_hbm, out_hbm, idx_v, rows_a, rows_b, sem_a, sem_b):
        wid = lax.axis_index("s") * 2 + lax.axis_index("c")
        base = wid * tpw
        pltpu.sync_copy(ids_hbm.at[pl.ds(base, tpw)], idx_v)
        bufs = (rows_a, rows_b)
        sems = (sem_a, sem_b)

        def gather(c):
            return pltpu.async_copy(
                table_hbm.at[idx_v.at[pl.ds(c * _CH, _CH)]], bufs[c % 2],
                sems[c % 2])

        dmas = [gather(0)]
        if nch > 1:
            dmas.append(gather(1))
        for c in range(nch):
            dmas[c].wait()
            pltpu.sync_copy(bufs[c % 2], out_hbm.at[pl.ds(base + c * _CH, _CH)])
            if c + 2 < nch:
                dmas.append(gather(c + 2))

    return k(table, ids)


def _seg_idx_const():
    o = jnp.arange(_NSEG, dtype=jnp.int32)
    cpr = o % (3 * _P)
    rest = o // (3 * _P)
    gc = rest % _G
    rest = rest // _G
    gr = rest % _G
    b = rest // _G
    c = cpr // _P
    pr = cpr % _P
    return ((b * 3 + c) * _HW + gr * _P + pr) * (_HW // _P) + gc


def _tc_vis(images, vboxes, spatial_W, spatial_b, patch_W, patch_b):
    def body(img_ref, vb_ref, sw_ref, sb_ref, pw_ref, pb_ref, out_ref):
        xp = (img_ref[0].reshape(3, _G, _P, _G, _P)
              .transpose(1, 3, 0, 2, 4).reshape(_V, 3 * _P * _P))
        out_ref[0] = (jnp.dot(xp, pw_ref[...],
                              preferred_element_type=jnp.float32) + pb_ref[...]
                      + jnp.dot(vb_ref[...], sw_ref[...],
                                preferred_element_type=jnp.float32) + sb_ref[...])

    return pl.pallas_call(
        body,
        grid=(_B,),
        in_specs=[
            pl.BlockSpec((1, 3, _HW, _HW), lambda b: (b, 0, 0, 0)),
            pl.BlockSpec((_V, 4), lambda b: (0, 0)),
            pl.BlockSpec((4, _D), lambda b: (0, 0)),
            pl.BlockSpec((_D,), lambda b: (0,)),
            pl.BlockSpec((3 * _P * _P, _D), lambda b: (0, 0)),
            pl.BlockSpec((_D,), lambda b: (0,)),
        ],
        out_specs=pl.BlockSpec((1, _V, _D), lambda b: (b, 0, 0)),
        out_shape=jax.ShapeDtypeStruct((_B, _V, _D), jnp.float32),
    )(images, vboxes, spatial_W, spatial_b, patch_W, patch_b)


def _tc_fuse_half(sem_h, boxes, vis, spatial_W, spatial_b, boff, prev=None):
    def body(sem_ref, boxes_ref, vis_ref, sw_ref, sb_ref, *rest):
        out_ref = rest[-1]
        j = pl.program_id(1)

        @pl.when(j < _NJ - 1)
        def _():
            out_ref[0] = (sem_ref[0]
                          + jnp.dot(boxes_ref[0], sw_ref[...],
                                    preferred_element_type=jnp.float32)
                          + sb_ref[...])

        @pl.when(j == _NJ - 1)
        def _():
            out_ref[0, :_V, :] = vis_ref[0]

    jmax = _NJ - 2
    in_specs = [
        pl.BlockSpec((1, _BLK, _D), lambda b, j: (b, jnp.minimum(j, jmax), 0)),
        pl.BlockSpec((1, _BLK, 4),
                     lambda b, j: (b + boff, jnp.minimum(j, jmax), 0)),
        pl.BlockSpec((1, _V, _D), lambda b, j: (b + boff, 0, 0)),
        pl.BlockSpec((4, _D), lambda b, j: (0, 0)),
        pl.BlockSpec((_D,), lambda b, j: (0,)),
    ]
    args = [sem_h, boxes, vis, spatial_W, spatial_b]
    aliases = {}
    if prev is not None:
        in_specs.append(pl.BlockSpec(memory_space=pl.ANY))
        args.append(prev)
        aliases = {5: 0}
    return pl.pallas_call(
        body,
        grid=(2, _NJ),
        in_specs=in_specs,
        out_specs=pl.BlockSpec((1, _BLK, _D), lambda b, j: (b + boff, j, 0)),
        out_shape=jax.ShapeDtypeStruct((_B, _SEQ, _D), jnp.float32),
        input_output_aliases=aliases,
    )(*args)


def _vbox_const():
    r = jnp.arange(_G, dtype=jnp.float32)
    c = jnp.arange(_G, dtype=jnp.float32)
    rr, cc = jnp.meshgrid(r, c, indexing='ij')
    x0 = (cc / _G).reshape(-1)
    y0 = (rr / _G).reshape(-1)
    x1 = ((cc + 1.0) / _G).reshape(-1)
    y1 = ((rr + 1.0) / _G).reshape(-1)
    return jnp.stack([x0, y0, x1, y1], axis=-1)


def kernel(input_ids, boxes, images, shared_table, spatial_W, spatial_b,
           patch_W, patch_b):
    ids = input_ids.reshape(-1).astype(jnp.int32)
    half = _TOK // 2
    sem_a = _sc_gather(shared_table, ids[:half], half)
    sem_b = _sc_gather(shared_table, ids[half:], half)
    vis = _tc_vis(images, _vbox_const(), spatial_W, spatial_b, patch_W, patch_b)
    out = _tc_fuse_half(sem_a.reshape(2, _S, _D), boxes, vis, spatial_W,
                        spatial_b, 0)
    return _tc_fuse_half(sem_b.reshape(2, _S, _D), boxes, vis, spatial_W,
                         spatial_b, 2, prev=out)


# speedup vs baseline: 1.0284x; 1.0284x over previous
import functools

import jax
import jax.numpy as jnp
from jax import lax
from jax.experimental import pallas as pl
from jax.experimental.pallas import tpu as pltpu
from jax.experimental.pallas import tpu_sc as plsc

_VOCAB = 100000
_D = 768
_B = 4
_S = 2048
_HW = 224
_P = 16
_G = _HW // _P
_V = _G * _G
_SEQ = _S + _V
_BLK = 256
_NJ = _SEQ // _BLK + 1

_NW = 32
_TOK = _B * _S
_TPW = _TOK // _NW
_CH = 64
_NCH = _TPW // _CH


def _sc_gather(table, ids):
    mesh = plsc.VectorSubcoreMesh(core_axis_name="c", subcore_axis_name="s")

    @functools.partial(
        pl.kernel,
        mesh=mesh,
        out_type=jax.ShapeDtypeStruct((_TOK, _D), jnp.float32),
        scratch_types=[
            pltpu.VMEM((_TPW,), jnp.int32),
            pltpu.VMEM((_CH, _D), jnp.float32),
            pltpu.VMEM((_CH, _D), jnp.float32),
            pltpu.SemaphoreType.DMA,
            pltpu.SemaphoreType.DMA,
        ],
    )
    def k(table_hbm, ids_hbm, out_hbm, idx_v, rows_a, rows_b, sem_a, sem_b):
        wid = lax.axis_index("s") * 2 + lax.axis_index("c")
        base = wid * _TPW
        pltpu.sync_copy(ids_hbm.at[pl.ds(base, _TPW)], idx_v)
        bufs = (rows_a, rows_b)
        sems = (sem_a, sem_b)

        def gather(c):
            return pltpu.async_copy(
                table_hbm.at[idx_v.at[pl.ds(c * _CH, _CH)]], bufs[c % 2],
                sems[c % 2])

        dmas = [gather(0), gather(1)]
        for c in range(_NCH):
            dmas[c].wait()
            pltpu.sync_copy(bufs[c % 2], out_hbm.at[pl.ds(base + c * _CH, _CH)])
            if c + 2 < _NCH:
                dmas.append(gather(c + 2))

    return k(table, ids)


def _tc_vis(images, vboxes, spatial_W, spatial_b, patch_W, patch_b):
    def body(img_ref, vb_ref, sw_ref, sb_ref, pw_ref, pb_ref, out_ref):
        xp = (img_ref[0].reshape(3, _G, _P, _G, _P)
              .transpose(1, 3, 0, 2, 4).reshape(_V, 3 * _P * _P))
        out_ref[0] = (jnp.dot(xp, pw_ref[...],
                              preferred_element_type=jnp.float32) + pb_ref[...]
                      + jnp.dot(vb_ref[...], sw_ref[...],
                                preferred_element_type=jnp.float32) + sb_ref[...])

    return pl.pallas_call(
        body,
        grid=(_B,),
        in_specs=[
            pl.BlockSpec((1, 3, _HW, _HW), lambda b: (b, 0, 0, 0)),
            pl.BlockSpec((_V, 4), lambda b: (0, 0)),
            pl.BlockSpec((4, _D), lambda b: (0, 0)),
            pl.BlockSpec((_D,), lambda b: (0,)),
            pl.BlockSpec((3 * _P * _P, _D), lambda b: (0, 0)),
            pl.BlockSpec((_D,), lambda b: (0,)),
        ],
        out_specs=pl.BlockSpec((1, _V, _D), lambda b: (b, 0, 0)),
        out_shape=jax.ShapeDtypeStruct((_B, _V, _D), jnp.float32),
    )(images, vboxes, spatial_W, spatial_b, patch_W, patch_b)


def _tc_fuse(sem, boxes, vis, spatial_W, spatial_b):
    def body(sem_ref, boxes_ref, vis_ref, sw_ref, sb_ref, out_ref):
        j = pl.program_id(1)

        @pl.when(j < _NJ - 1)
        def _():
            out_ref[0] = (sem_ref[0]
                          + jnp.dot(boxes_ref[0], sw_ref[...],
                                    preferred_element_type=jnp.float32)
                          + sb_ref[...])

        @pl.when(j == _NJ - 1)
        def _():
            out_ref[0, :_V, :] = vis_ref[0]

    jmax = _NJ - 2
    return pl.pallas_call(
        body,
        grid=(_B, _NJ),
        in_specs=[
            pl.BlockSpec((1, _BLK, _D), lambda b, j: (b, jnp.minimum(j, jmax), 0)),
            pl.BlockSpec((1, _BLK, 4), lambda b, j: (b, jnp.minimum(j, jmax), 0)),
            pl.BlockSpec((1, _V, _D), lambda b, j: (b, 0, 0)),
            pl.BlockSpec((4, _D), lambda b, j: (0, 0)),
            pl.BlockSpec((_D,), lambda b, j: (0,)),
        ],
        out_specs=pl.BlockSpec((1, _BLK, _D), lambda b, j: (b, j, 0)),
        out_shape=jax.ShapeDtypeStruct((_B, _SEQ, _D), jnp.float32),
    )(sem, boxes, vis, spatial_W, spatial_b)


def _vbox_const():
    r = jnp.arange(_G, dtype=jnp.float32)
    c = jnp.arange(_G, dtype=jnp.float32)
    rr, cc = jnp.meshgrid(r, c, indexing='ij')
    x0 = (cc / _G).reshape(-1)
    y0 = (rr / _G).reshape(-1)
    x1 = ((cc + 1.0) / _G).reshape(-1)
    y1 = ((rr + 1.0) / _G).reshape(-1)
    return jnp.stack([x0, y0, x1, y1], axis=-1)


def kernel(input_ids, boxes, images, shared_table, spatial_W, spatial_b,
           patch_W, patch_b):
    ids = input_ids.reshape(-1).astype(jnp.int32)
    sem = _sc_gather(shared_table, ids)
    vis = _tc_vis(images, _vbox_const(), spatial_W, spatial_b, patch_W, patch_b)
    return _tc_fuse(sem.reshape(_B, _S, _D), boxes, vis, spatial_W, spatial_b)


# speedup vs baseline: 1.1384x; 1.1069x over previous
import functools

import jax
import jax.numpy as jnp
from jax import lax
from jax.experimental import pallas as pl
from jax.experimental.pallas import tpu as pltpu
from jax.experimental.pallas import tpu_sc as plsc

_VOCAB = 100000
_D = 768
_B = 4
_S = 2048
_HW = 224
_P = 16
_G = _HW // _P
_V = _G * _G
_SEQ = _S + _V
_BLK = 512
_NJ = _SEQ // _BLK + 1

_NW = 32
_TOK = _B * _S
_TPW = _TOK // _NW
_CH = 64
_NCH = _TPW // _CH


def _sc_gather(table, ids):
    mesh = plsc.VectorSubcoreMesh(core_axis_name="c", subcore_axis_name="s")

    @functools.partial(
        pl.kernel,
        mesh=mesh,
        out_type=jax.ShapeDtypeStruct((_TOK, _D), jnp.float32),
        scratch_types=[
            pltpu.VMEM((_TPW,), jnp.int32),
            pltpu.VMEM((_CH, _D), jnp.float32),
            pltpu.VMEM((_CH, _D), jnp.float32),
            pltpu.SemaphoreType.DMA,
            pltpu.SemaphoreType.DMA,
        ],
    )
    def k(table_hbm, ids_hbm, out_hbm, idx_v, rows_a, rows_b, sem_a, sem_b):
        wid = lax.axis_index("s") * 2 + lax.axis_index("c")
        base = wid * _TPW
        pltpu.sync_copy(ids_hbm.at[pl.ds(base, _TPW)], idx_v)
        bufs = (rows_a, rows_b)
        sems = (sem_a, sem_b)

        def gather(c):
            return pltpu.async_copy(
                table_hbm.at[idx_v.at[pl.ds(c * _CH, _CH)]], bufs[c % 2],
                sems[c % 2])

        dmas = [gather(0), gather(1)]
        for c in range(_NCH):
            dmas[c].wait()
            pltpu.sync_copy(bufs[c % 2], out_hbm.at[pl.ds(base + c * _CH, _CH)])
            if c + 2 < _NCH:
                dmas.append(gather(c + 2))

    return k(table, ids)


def _tc_vis(images, vboxes, spatial_W, spatial_b, patch_W, patch_b):
    def body(img_ref, vb_ref, sw_ref, sb_ref, pw_ref, pb_ref, out_ref):
        xp = (img_ref[0].reshape(3, _G, _P, _G, _P)
              .transpose(1, 3, 0, 2, 4).reshape(_V, 3 * _P * _P))
        out_ref[0] = (jnp.dot(xp, pw_ref[...],
                              preferred_element_type=jnp.float32) + pb_ref[...]
                      + jnp.dot(vb_ref[...], sw_ref[...],
                                preferred_element_type=jnp.float32) + sb_ref[...])

    return pl.pallas_call(
        body,
        grid=(_B,),
        in_specs=[
            pl.BlockSpec((1, 3, _HW, _HW), lambda b: (b, 0, 0, 0)),
            pl.BlockSpec((_V, 4), lambda b: (0, 0)),
            pl.BlockSpec((4, _D), lambda b: (0, 0)),
            pl.BlockSpec((_D,), lambda b: (0,)),
            pl.BlockSpec((3 * _P * _P, _D), lambda b: (0, 0)),
            pl.BlockSpec((_D,), lambda b: (0,)),
        ],
        out_specs=pl.BlockSpec((1, _V, _D), lambda b: (b, 0, 0)),
        out_shape=jax.ShapeDtypeStruct((_B, _V, _D), jnp.float32),
    )(images, vboxes, spatial_W, spatial_b, patch_W, patch_b)


def _tc_fuse(sem, boxes, vis, spatial_W, spatial_b):
    def body(sem_ref, boxes_ref, vis_ref, sw_ref, sb_ref, out_ref):
        j = pl.program_id(1)

        @pl.when(j < _NJ - 1)
        def _():
            out_ref[0] = (sem_ref[0]
                          + jnp.dot(boxes_ref[0], sw_ref[...],
                                    preferred_element_type=jnp.float32)
                          + sb_ref[...])

        @pl.when(j == _NJ - 1)
        def _():
            out_ref[0, :_V, :] = vis_ref[0]

    jmax = _NJ - 2
    return pl.pallas_call(
        body,
        grid=(_B, _NJ),
        in_specs=[
            pl.BlockSpec((1, _BLK, _D), lambda b, j: (b, jnp.minimum(j, jmax), 0)),
            pl.BlockSpec((1, _BLK, 4), lambda b, j: (b, jnp.minimum(j, jmax), 0)),
            pl.BlockSpec((1, _V, _D), lambda b, j: (b, 0, 0)),
            pl.BlockSpec((4, _D), lambda b, j: (0, 0)),
            pl.BlockSpec((_D,), lambda b, j: (0,)),
        ],
        out_specs=pl.BlockSpec((1, _BLK, _D), lambda b, j: (b, j, 0)),
        out_shape=jax.ShapeDtypeStruct((_B, _SEQ, _D), jnp.float32),
    )(sem, boxes, vis, spatial_W, spatial_b)


def _vbox_const():
    r = jnp.arange(_G, dtype=jnp.float32)
    c = jnp.arange(_G, dtype=jnp.float32)
    rr, cc = jnp.meshgrid(r, c, indexing='ij')
    x0 = (cc / _G).reshape(-1)
    y0 = (rr / _G).reshape(-1)
    x1 = ((cc + 1.0) / _G).reshape(-1)
    y1 = ((rr + 1.0) / _G).reshape(-1)
    return jnp.stack([x0, y0, x1, y1], axis=-1)


def kernel(input_ids, boxes, images, shared_table, spatial_W, spatial_b,
           patch_W, patch_b):
    ids = input_ids.reshape(-1).astype(jnp.int32)
    sem = _sc_gather(shared_table, ids)
    vis = _tc_vis(images, _vbox_const(), spatial_W, spatial_b, patch_W, patch_b)
    return _tc_fuse(sem.reshape(_B, _S, _D), boxes, vis, spatial_W, spatial_b)


# speedup vs baseline: 1.1953x; 1.0499x over previous
import functools

import jax
import jax.numpy as jnp
from jax import lax
from jax.experimental import pallas as pl
from jax.experimental.pallas import tpu as pltpu
from jax.experimental.pallas import tpu_sc as plsc

_VOCAB = 100000
_D = 768
_B = 4
_S = 2048
_HW = 224
_P = 16
_G = _HW // _P
_V = _G * _G
_SEQ = _S + _V
_BLK = 1024
_NJ = _SEQ // _BLK + 1

_NW = 32
_TOK = _B * _S
_TPW = _TOK // _NW
_CH = 64
_NCH = _TPW // _CH


def _sc_gather(table, ids):
    mesh = plsc.VectorSubcoreMesh(core_axis_name="c", subcore_axis_name="s")

    @functools.partial(
        pl.kernel,
        mesh=mesh,
        out_type=jax.ShapeDtypeStruct((_TOK, _D), jnp.float32),
        scratch_types=[
            pltpu.VMEM((_TPW,), jnp.int32),
            pltpu.VMEM((_CH, _D), jnp.float32),
            pltpu.VMEM((_CH, _D), jnp.float32),
            pltpu.SemaphoreType.DMA,
            pltpu.SemaphoreType.DMA,
        ],
    )
    def k(table_hbm, ids_hbm, out_hbm, idx_v, rows_a, rows_b, sem_a, sem_b):
        wid = lax.axis_index("s") * 2 + lax.axis_index("c")
        base = wid * _TPW
        pltpu.sync_copy(ids_hbm.at[pl.ds(base, _TPW)], idx_v)
        bufs = (rows_a, rows_b)
        sems = (sem_a, sem_b)

        def gather(c):
            return pltpu.async_copy(
                table_hbm.at[idx_v.at[pl.ds(c * _CH, _CH)]], bufs[c % 2],
                sems[c % 2])

        dmas = [gather(0), gather(1)]
        for c in range(_NCH):
            dmas[c].wait()
            pltpu.sync_copy(bufs[c % 2], out_hbm.at[pl.ds(base + c * _CH, _CH)])
            if c + 2 < _NCH:
                dmas.append(gather(c + 2))

    return k(table, ids)


def _tc_vis(images, vboxes, spatial_W, spatial_b, patch_W, patch_b):
    def body(img_ref, vb_ref, sw_ref, sb_ref, pw_ref, pb_ref, out_ref):
        xp = (img_ref[0].reshape(3, _G, _P, _G, _P)
              .transpose(1, 3, 0, 2, 4).reshape(_V, 3 * _P * _P))
        out_ref[0] = (jnp.dot(xp, pw_ref[...],
                              preferred_element_type=jnp.float32) + pb_ref[...]
                      + jnp.dot(vb_ref[...], sw_ref[...],
                                preferred_element_type=jnp.float32) + sb_ref[...])

    return pl.pallas_call(
        body,
        grid=(_B,),
        in_specs=[
            pl.BlockSpec((1, 3, _HW, _HW), lambda b: (b, 0, 0, 0)),
            pl.BlockSpec((_V, 4), lambda b: (0, 0)),
            pl.BlockSpec((4, _D), lambda b: (0, 0)),
            pl.BlockSpec((_D,), lambda b: (0,)),
            pl.BlockSpec((3 * _P * _P, _D), lambda b: (0, 0)),
            pl.BlockSpec((_D,), lambda b: (0,)),
        ],
        out_specs=pl.BlockSpec((1, _V, _D), lambda b: (b, 0, 0)),
        out_shape=jax.ShapeDtypeStruct((_B, _V, _D), jnp.float32),
    )(images, vboxes, spatial_W, spatial_b, patch_W, patch_b)


def _tc_fuse(sem, boxes, vis, spatial_W, spatial_b):
    def body(sem_ref, boxes_ref, vis_ref, sw_ref, sb_ref, out_ref):
        j = pl.program_id(1)

        @pl.when(j < _NJ - 1)
        def _():
            out_ref[0] = (sem_ref[0]
                          + jnp.dot(boxes_ref[0], sw_ref[...],
                                    preferred_element_type=jnp.float32)
                          + sb_ref[...])

        @pl.when(j == _NJ - 1)
        def _():
            out_ref[0, :_V, :] = vis_ref[0]

    jmax = _NJ - 2
    return pl.pallas_call(
        body,
        grid=(_B, _NJ),
        in_specs=[
            pl.BlockSpec((1, _BLK, _D), lambda b, j: (b, jnp.minimum(j, jmax), 0)),
            pl.BlockSpec((1, _BLK, 4), lambda b, j: (b, jnp.minimum(j, jmax), 0)),
            pl.BlockSpec((1, _V, _D), lambda b, j: (b, 0, 0)),
            pl.BlockSpec((4, _D), lambda b, j: (0, 0)),
            pl.BlockSpec((_D,), lambda b, j: (0,)),
        ],
        out_specs=pl.BlockSpec((1, _BLK, _D), lambda b, j: (b, j, 0)),
        out_shape=jax.ShapeDtypeStruct((_B, _SEQ, _D), jnp.float32),
    )(sem, boxes, vis, spatial_W, spatial_b)


def _vbox_const():
    r = jnp.arange(_G, dtype=jnp.float32)
    c = jnp.arange(_G, dtype=jnp.float32)
    rr, cc = jnp.meshgrid(r, c, indexing='ij')
    x0 = (cc / _G).reshape(-1)
    y0 = (rr / _G).reshape(-1)
    x1 = ((cc + 1.0) / _G).reshape(-1)
    y1 = ((rr + 1.0) / _G).reshape(-1)
    return jnp.stack([x0, y0, x1, y1], axis=-1)


def kernel(input_ids, boxes, images, shared_table, spatial_W, spatial_b,
           patch_W, patch_b):
    ids = input_ids.reshape(-1).astype(jnp.int32)
    sem = _sc_gather(shared_table, ids)
    vis = _tc_vis(images, _vbox_const(), spatial_W, spatial_b, patch_W, patch_b)
    return _tc_fuse(sem.reshape(_B, _S, _D), boxes, vis, spatial_W, spatial_b)
